# baseline (speedup 1.0000x reference)
import jax
import jax.numpy as jnp
from jax.experimental import pallas as pl
from jax.experimental.pallas import tpu as pltpu

_CM = 400
_NBUF = 3


def _chunk_copy(adj_ref, abuf, sems, chunk):
    slot = jax.lax.rem(chunk, _NBUF)
    return pltpu.make_async_copy(
        adj_ref.at[pl.ds(chunk * _CM, _CM), :],
        abuf.at[slot],
        sems.at[slot],
    )


def _ngcf_fused(adj_ref, x_ref, w1_ref, w2_ref, b1_ref, b2_ref,
                out_ref, abuf, xbf_s, w1t_s, w2t_s, sems):
    i = pl.program_id(0)
    nsteps = pl.num_programs(0)

    @pl.when(i == 0)
    def _init():
        for c in range(_NBUF):
            _chunk_copy(adj_ref, abuf, sems, c).start()
        xbf_s[...] = x_ref[...].astype(jnp.bfloat16)
        w1t_s[...] = w1_ref[...].T.astype(jnp.bfloat16)
        w2t_s[...] = w2_ref[...].T.astype(jnp.bfloat16)

    @pl.when(jnp.logical_and(i > 0, i + _NBUF - 1 < nsteps))
    def _refill():
        _chunk_copy(adj_ref, abuf, sems, i - 1 + _NBUF).start()

    _chunk_copy(adj_ref, abuf, sems, i).wait()
    slot = jax.lax.rem(i, _NBUF)
    n = x_ref.shape[0]
    nk = 4
    ks = n // nk
    neigh = jnp.zeros((_CM, x_ref.shape[1]), jnp.float32)
    for k in range(nk):
        a = abuf[slot][:, k * ks:(k + 1) * ks].astype(jnp.bfloat16)
        neigh += jnp.dot(a, xbf_s[k * ks:(k + 1) * ks, :],
                         preferred_element_type=jnp.float32)
    xblk = x_ref[pl.ds(i * _CM, _CM), :]
    inter = xblk * neigh
    h = (jnp.dot(neigh.astype(jnp.bfloat16), w1t_s[...],
                 preferred_element_type=jnp.float32)
         + jnp.dot(inter.astype(jnp.bfloat16), w2t_s[...],
                   preferred_element_type=jnp.float32)
         + b1_ref[...] + b2_ref[...])
    out_ref[...] = jnp.where(h >= 0, h, 0.2 * h)


def kernel(x, adj_matrix, W1, b1, W2, b2):
    n, d = x.shape
    d_out = W1.shape[0]
    grid = (n // _CM,)
    return pl.pallas_call(
        _ngcf_fused,
        grid=grid,
        in_specs=[
            pl.BlockSpec(memory_space=pl.ANY),
            pl.BlockSpec((n, d), lambda i: (0, 0)),
            pl.BlockSpec((d_out, d), lambda i: (0, 0)),
            pl.BlockSpec((d_out, d), lambda i: (0, 0)),
            pl.BlockSpec((1, d_out), lambda i: (0, 0)),
            pl.BlockSpec((1, d_out), lambda i: (0, 0)),
        ],
        out_specs=pl.BlockSpec((_CM, d_out), lambda i: (i, 0)),
        out_shape=jax.ShapeDtypeStruct((n, d_out), jnp.float32),
        scratch_shapes=[
            pltpu.VMEM((_NBUF, _CM, n), jnp.float32),
            pltpu.VMEM((n, d), jnp.bfloat16),
            pltpu.VMEM((d, d_out), jnp.bfloat16),
            pltpu.VMEM((d, d_out), jnp.bfloat16),
            pltpu.SemaphoreType.DMA((_NBUF,)),
        ],
        compiler_params=pltpu.CompilerParams(
            dimension_semantics=("arbitrary",),
            vmem_limit_bytes=100 * 1024 * 1024,
        ),
    )(adj_matrix, x, W1, W2, b1.reshape(1, -1), b2.reshape(1, -1))


# speedup vs baseline: 1.0341x; 1.0341x over previous
import jax
import jax.numpy as jnp
from jax.experimental import pallas as pl
from jax.experimental.pallas import tpu as pltpu

_BM = 200


def _ngcf_fused(adja_ref, adjb_ref, x_ref, w1_ref, w2_ref, b1_ref, b2_ref,
                out_ref, xbf_s, w1t_s, w2t_s):
    i = pl.program_id(0)

    @pl.when(i == 0)
    def _init():
        xbf_s[...] = x_ref[...].astype(jnp.bfloat16)
        w1t_s[...] = w1_ref[...].T.astype(jnp.bfloat16)
        w2t_s[...] = w2_ref[...].T.astype(jnp.bfloat16)

    def half(adj_ref, row0):
        a = adj_ref[...].astype(jnp.bfloat16)
        neigh = jnp.dot(a, xbf_s[...], preferred_element_type=jnp.float32)
        xblk = x_ref[pl.ds(row0, _BM), :]
        inter = xblk * neigh
        h = (jnp.dot(neigh.astype(jnp.bfloat16), w1t_s[...],
                     preferred_element_type=jnp.float32)
             + jnp.dot(inter.astype(jnp.bfloat16), w2t_s[...],
                       preferred_element_type=jnp.float32)
             + b1_ref[...] + b2_ref[...])
        return jnp.where(h >= 0, h, 0.2 * h)

    out_ref[0:_BM, :] = half(adja_ref, 2 * i * _BM)
    out_ref[_BM:2 * _BM, :] = half(adjb_ref, (2 * i + 1) * _BM)


def kernel(x, adj_matrix, W1, b1, W2, b2):
    n, d = x.shape
    d_out = W1.shape[0]
    grid = (n // (2 * _BM),)
    return pl.pallas_call(
        _ngcf_fused,
        grid=grid,
        in_specs=[
            pl.BlockSpec((_BM, n), lambda i: (2 * i, 0)),
            pl.BlockSpec((_BM, n), lambda i: (2 * i + 1, 0)),
            pl.BlockSpec((n, d), lambda i: (0, 0)),
            pl.BlockSpec((d_out, d), lambda i: (0, 0)),
            pl.BlockSpec((d_out, d), lambda i: (0, 0)),
            pl.BlockSpec((1, d_out), lambda i: (0, 0)),
            pl.BlockSpec((1, d_out), lambda i: (0, 0)),
        ],
        out_specs=pl.BlockSpec((2 * _BM, d_out), lambda i: (i, 0)),
        out_shape=jax.ShapeDtypeStruct((n, d_out), jnp.float32),
        scratch_shapes=[
            pltpu.VMEM((n, d), jnp.bfloat16),
            pltpu.VMEM((d, d_out), jnp.bfloat16),
            pltpu.VMEM((d, d_out), jnp.bfloat16),
        ],
        compiler_params=pltpu.CompilerParams(
            dimension_semantics=("arbitrary",),
            vmem_limit_bytes=100 * 1024 * 1024,
        ),
    )(adj_matrix, adj_matrix, x, W1, W2, b1.reshape(1, -1), b2.reshape(1, -1))


# speedup vs baseline: 1.0389x; 1.0047x over previous
import jax
import jax.numpy as jnp
from jax.experimental import pallas as pl
from jax.experimental.pallas import tpu as pltpu

_BM = 400


def _ngcf_fused(adj_ref, x_ref, w1_ref, w2_ref, b1_ref, b2_ref,
                out_ref, xbf_s, w1t_s, w2t_s):
    i = pl.program_id(0)

    @pl.when(i == 0)
    def _init():
        xbf_s[...] = x_ref[...].astype(jnp.bfloat16)
        w1t_s[...] = w1_ref[...].T.astype(jnp.bfloat16)
        w2t_s[...] = w2_ref[...].T.astype(jnp.bfloat16)

    a = adj_ref[...].astype(jnp.bfloat16)
    neigh = jnp.dot(a, xbf_s[...], preferred_element_type=jnp.float32)
    xblk = x_ref[pl.ds(i * _BM, _BM), :]
    inter = xblk * neigh
    h = (jnp.dot(neigh.astype(jnp.bfloat16), w1t_s[...],
                 preferred_element_type=jnp.float32)
         + jnp.dot(inter.astype(jnp.bfloat16), w2t_s[...],
                   preferred_element_type=jnp.float32)
         + b1_ref[...] + b2_ref[...])
    out_ref[...] = jnp.where(h >= 0, h, 0.2 * h)


def kernel(x, adj_matrix, W1, b1, W2, b2):
    n, d = x.shape
    d_out = W1.shape[0]
    grid = (n // _BM,)
    return pl.pallas_call(
        _ngcf_fused,
        grid=grid,
        in_specs=[
            pl.BlockSpec((_BM, n), lambda i: (i, 0)),
            pl.BlockSpec((n, d), lambda i: (0, 0)),
            pl.BlockSpec((d_out, d), lambda i: (0, 0)),
            pl.BlockSpec((d_out, d), lambda i: (0, 0)),
            pl.BlockSpec((1, d_out), lambda i: (0, 0)),
            pl.BlockSpec((1, d_out), lambda i: (0, 0)),
        ],
        out_specs=pl.BlockSpec((_BM, d_out), lambda i: (i, 0)),
        out_shape=jax.ShapeDtypeStruct((n, d_out), jnp.float32),
        scratch_shapes=[
            pltpu.VMEM((n, d), jnp.bfloat16),
            pltpu.VMEM((d, d_out), jnp.bfloat16),
            pltpu.VMEM((d, d_out), jnp.bfloat16),
        ],
        compiler_params=pltpu.CompilerParams(
            dimension_semantics=("arbitrary",),
            vmem_limit_bytes=100 * 1024 * 1024,
        ),
    )(adj_matrix, x, W1, W2, b1.reshape(1, -1), b2.reshape(1, -1))
